# baseline (speedup 1.0000x reference)
import functools

import jax
import jax.numpy as jnp
from jax import lax
from jax.experimental import pallas as pl
from jax.experimental.pallas import tpu as pltpu
from jax.experimental.pallas import tpu_sc as plsc

LANES = 16


def _ds8(start, size):
    return pl.ds(pl.multiple_of(start, 8), size)


@functools.lru_cache(maxsize=None)
def _build(total, batch, n_fields, d, n_workers, n_cores):
    rows_per_worker = total // n_workers
    chunk_elems = 4
    chunk_rows = chunk_elems * n_fields
    n_chunks = rows_per_worker // chunk_rows
    pat = 208
    assert rows_per_worker % pat == 0
    d_vecs = d // LANES

    mesh = plsc.VectorSubcoreMesh(core_axis_name="c", subcore_axis_name="s")

    @functools.partial(
        pl.kernel,
        mesh=mesh,
        out_type=jax.ShapeDtypeStruct((total // 2, 2 * d), jnp.float32),
        scratch_types=[
            pltpu.VMEM((rows_per_worker,), jnp.int32),
            pltpu.VMEM((rows_per_worker,), jnp.int32),
            pltpu.VMEM((pat,), jnp.int32),
            pltpu.VMEM((n_fields * d,), jnp.float32),
            pltpu.VMEM((chunk_rows, 2 * d), jnp.float32),
            pltpu.VMEM((chunk_rows, 2 * d), jnp.float32),
            pltpu.SemaphoreType.DMA,
        ],
        compiler_params=pltpu.CompilerParams(needs_layout_passes=False),
    )
    def emb_kernel(x_hbm, off_hbm, table_hbm, bias_hbm, out_hbm,
                   idx_v, par_v, off_v, bias_v, wide_v, out_v, sem):
        cid = lax.axis_index("c")
        sid = lax.axis_index("s")
        wid = sid * n_cores + cid
        wbase = wid * rows_per_worker

        pltpu.sync_copy(x_hbm.at[_ds8(wbase, rows_per_worker)], idx_v)
        pltpu.sync_copy(off_hbm, off_v)
        pltpu.sync_copy(bias_hbm, bias_v)

        iota = lax.iota(jnp.int32, LANES)

        def prep(i, carry):
            base = i * pat
            for j in range(pat // LANES):
                sl = _ds8(base + j * LANES, LANES)
                g = idx_v[sl] + off_v[pl.ds(j * LANES, LANES)]
                idx_v[sl] = lax.shift_right_logical(g, 1)
                par_v[sl] = lax.shift_left(jnp.bitwise_and(g, 1), 6)
            return carry
        lax.fori_loop(0, rows_per_worker // pat, prep, 0)

        def pair_body(t, carry):
            for p in range(2):
                cbase = (2 * t + p) * chunk_rows
                pltpu.async_copy(
                    table_hbm.at[idx_v.at[_ds8(cbase, chunk_rows)]],
                    wide_v, sem).wait()
                orow0 = p * (chunk_rows // 2)
                for r in range(chunk_rows):
                    f = r % n_fields
                    lane = jnp.broadcast_to(cbase + r, (LANES,))
                    half = plsc.load_gather(par_v, [lane])
                    colbase = half + iota
                    rsel = jnp.broadcast_to(jnp.int32(r), (LANES,))
                    for j in range(d_vecs):
                        v = plsc.load_gather(
                            wide_v, [rsel, colbase + (j * LANES)])
                        out_v[orow0 + r // 2,
                              pl.ds((r % 2) * d + j * LANES, LANES)] = (
                            v + bias_v[pl.ds(f * d + j * LANES, LANES)])
            pltpu.sync_copy(
                out_v,
                out_hbm.at[_ds8((wbase + 2 * t * chunk_rows) // 2,
                                chunk_rows)])
            return carry
        lax.fori_loop(0, n_chunks // 2, pair_body, 0)

    return emb_kernel


def kernel(x, table, bias, offsets):
    batch, n_fields = x.shape
    v, d = table.shape
    total = batch * n_fields

    info = plsc.get_sparse_core_info()
    n_workers = info.num_cores * info.num_subcores

    x_flat = x.reshape(-1).astype(jnp.int32)
    off_rep = jnp.tile(offsets.astype(jnp.int32), 208 // n_fields)
    bias_flat = bias.reshape(-1)
    table_wide = table.reshape(v // 2, 2 * d)

    out = _build(total, batch, n_fields, d, n_workers, info.num_cores)(
        x_flat, off_rep, table_wide, bias_flat)
    return out.reshape(batch, n_fields, d)


# speedup vs baseline: 3.6048x; 3.6048x over previous
import functools

import jax
import jax.numpy as jnp
from jax import lax
from jax.experimental import pallas as pl
from jax.experimental.pallas import tpu as pltpu
from jax.experimental.pallas import tpu_sc as plsc

LANES = 16
NBUF = 4


def _dsa(start, size, align):
    return pl.ds(pl.multiple_of(start, align), size)


def _splat(x):
    return jnp.broadcast_to(jnp.asarray(x, jnp.int32), (LANES,))


@functools.lru_cache(maxsize=None)
def _build(batch, n_fields, v_rows, d, n_cores):
    total = batch * n_fields
    n_vec = batch // LANES
    hist_n = 1024
    d_vecs = d // LANES
    blocks = batch // 128

    mesh = plsc.VectorSubcoreMesh(core_axis_name="c", subcore_axis_name="s")

    @functools.partial(
        pl.kernel,
        mesh=mesh,
        out_type=jax.ShapeDtypeStruct((total, 128), jnp.float32),
        scratch_types=[
            pltpu.VMEM((batch,), jnp.int32),
            pltpu.VMEM((batch,), jnp.int32),
            pltpu.VMEM((batch,), jnp.int32),
            pltpu.VMEM((batch,), jnp.int32),
            pltpu.VMEM((hist_n,), jnp.int32),
            pltpu.VMEM((hist_n,), jnp.int32),
            pltpu.VMEM((blocks, 128), jnp.int32),
            pltpu.VMEM((n_fields,), jnp.int32),
            pltpu.VMEM((n_fields * d,), jnp.float32),
            pltpu.VMEM((NBUF, d, 128), jnp.float32),
            pltpu.VMEM((128, 128), jnp.float32),
            pltpu.SemaphoreType.DMA,
            pltpu.SemaphoreType.DMA,
            pltpu.SemaphoreType.DMA,
            pltpu.SemaphoreType.DMA,
            pltpu.SemaphoreType.DMA,
        ],
        compiler_params=pltpu.CompilerParams(needs_layout_passes=False),
    )
    def emb_kernel(xt_hbm, off_hbm, tbl_hbm, tail_hbm, bias_hbm, stage_hbm,
                   g_v, s_v, srt_g, srt_b, pref_v, run_v, oid_v, off_v,
                   bias_v, slab_v, outb_v, sem0, sem1, sem2, sem3, ssem):
        sems = [sem0, sem1, sem2, sem3]
        cid = lax.axis_index("c")
        sid = lax.axis_index("s")
        wid = sid * n_cores + cid

        @pl.when(wid < n_fields)
        def _worker():
            iota = lax.iota(jnp.int32, LANES)
            lane0 = iota == 0

            pltpu.sync_copy(xt_hbm.at[wid], g_v)
            pltpu.sync_copy(off_hbm, off_v)
            pltpu.sync_copy(bias_hbm, bias_v)

            offv = plsc.load_gather(off_v, [_splat(wid)])
            off_s = jnp.max(offv)
            nxtv = plsc.load_gather(
                off_v, [_splat(jnp.minimum(wid + 1, n_fields - 1))])
            end_g = jnp.where(wid + 1 < n_fields, jnp.max(nxtv),
                              jnp.int32(v_rows))
            first_slab = lax.shift_right_logical(off_s, 7)
            last_slab = lax.shift_right_logical(end_g - 1, 7)
            nslab = last_slab - first_slab + 1
            has_tail = (last_slab + 1) * 128 > v_rows

            bvs = [plsc.load_gather(bias_v, [_splat(wid * d) + iota + jb * LANES])
                   for jb in range(d_vecs)]

            def prep(i, c):
                sl = _dsa(i * LANES, LANES, 8)
                g = g_v[sl] + offv
                g_v[sl] = g
                s_v[sl] = lax.shift_right_logical(g, 7) - first_slab
                return c
            lax.fori_loop(0, n_vec, prep, 0)

            for k in range(hist_n // LANES):
                pref_v[pl.ds(k * LANES, LANES)] = jnp.zeros((LANES,), jnp.int32)

            def _rank_total(i):
                sl = _dsa(i * LANES, LANES, 8)
                sv = s_v[sl]
                rank = jnp.zeros((LANES,), jnp.int32)
                total = jnp.ones((LANES,), jnp.int32)
                hi_idx = jnp.int32(batch - 1)
                for k in range(1, LANES):
                    mb = iota >= k
                    idxb = jnp.clip(_splat(i * LANES - k) + iota, 0, hi_idx)
                    shb = plsc.load_gather(s_v, [idxb], mask=mb)
                    eqb = jnp.where(jnp.logical_and(mb, shb == sv), 1, 0)
                    rank = rank + eqb
                    mf = iota < LANES - k
                    idxf = jnp.clip(_splat(i * LANES + k) + iota, 0, hi_idx)
                    shf = plsc.load_gather(s_v, [idxf], mask=mf)
                    eqf = jnp.where(jnp.logical_and(mf, shf == sv), 1, 0)
                    total = total + eqb + eqf
                return sv, rank, total

            def hist(i, c):
                sv, rank, total = _rank_total(i)
                sv = jnp.clip(sv, 0, hist_n - 2)
                base = plsc.load_gather(pref_v, [sv])
                plsc.store_scatter(pref_v, [sv], base + total)
                return c
            lax.fori_loop(0, n_vec, hist, 0)

            carry = jnp.int32(0)
            for k in range(hist_n // LANES):
                sl = pl.ds(k * LANES, LANES)
                h = pref_v[sl]
                inc = plsc.cumsum(h)
                excl = inc - h + jnp.broadcast_to(carry, (LANES,))
                pref_v[sl] = excl
                run_v[sl] = excl
                carry = carry + jnp.max(inc)

            def rank_pass(i, c):
                sl = _dsa(i * LANES, LANES, 8)
                sv, rank, total = _rank_total(i)
                sv = jnp.clip(sv, 0, hist_n - 2)
                base = plsc.load_gather(run_v, [sv])
                slot = jnp.clip(base + rank, 0, batch - 1)
                plsc.store_scatter(srt_g, [slot], g_v[sl])
                plsc.store_scatter(srt_b, [slot], _splat(i * LANES) + iota)
                plsc.store_scatter(run_v, [sv], base + total)
                return c
            lax.fori_loop(0, n_vec, rank_pass, 0)

            def oid(j, c):
                for k in range(128 // LANES):
                    bv = srt_b[_dsa(j * 128 + k * LANES, LANES, 8)]
                    oid_v[j, pl.ds(k * LANES, LANES)] = jnp.clip(
                        bv * n_fields + _splat(wid), 0, total - 1)
                return c
            lax.fori_loop(0, blocks, oid, 0)

            def issue(s_idx, p):
                @pl.when(s_idx < nslab)
                def _():
                    is_tail = jnp.logical_and(has_tail, s_idx == nslab - 1)

                    @pl.when(is_tail)
                    def _():
                        pltpu.async_copy(tail_hbm, slab_v.at[p], sems[p])

                    @pl.when(jnp.logical_not(is_tail))
                    def _():
                        col0 = (first_slab + s_idx) * 128
                        pltpu.async_copy(
                            tbl_hbm.at[:, _dsa(col0, 128, 128)],
                            slab_v.at[p], sems[p])

            for p in range(NBUF):
                issue(jnp.int32(p), p)

            def sweep(t, c):
                for p in range(NBUF):
                    s_idx = t * NBUF + p

                    @pl.when(s_idx < nslab)
                    def _(p=p, s_idx=s_idx):
                        pltpu.make_async_copy(
                            tbl_hbm.at[:, _dsa(0, 128, 128)],
                            slab_v.at[p], sems[p]).wait()
                        hi = jnp.minimum(
                            jnp.max(plsc.load_gather(pref_v, [_splat(s_idx + 1)])),
                            jnp.int32(batch))
                        lo = jnp.minimum(
                            jnp.max(plsc.load_gather(pref_v, [_splat(s_idx)])), hi)

                        def occ(q):
                            gv = plsc.load_gather(
                                srt_g, [jnp.clip(_splat(q), 0, batch - 1)])
                            lanev = jnp.bitwise_and(gv, 127)
                            row = jnp.bitwise_and(q, 127)
                            for jb in range(d_vecs):
                                val = plsc.load_gather(
                                    slab_v.at[p], [iota + jb * LANES, lanev])
                                outb_v[row, pl.ds(jb * LANES, LANES)] = \
                                    val + bvs[jb]

                            @pl.when(row == 127)
                            def _():
                                blk = lax.shift_right_logical(q, 7)
                                pltpu.async_copy(
                                    outb_v, stage_hbm.at[oid_v.at[blk]],
                                    ssem).wait()
                            return q + 1
                        lax.while_loop(lambda q: q < hi, occ, lo)
                        issue(s_idx + NBUF, p)
                return c
            tmax = (nslab + NBUF - 1) // NBUF
            lax.fori_loop(0, tmax, sweep, 0)

    return emb_kernel


def kernel(x, table, bias, offsets):
    batch, n_fields = x.shape
    v_rows, d = table.shape

    info = plsc.get_sparse_core_info()

    xt = x.T.astype(jnp.int32)
    tbl_t = table.T
    tail_w = v_rows % 128
    tail_base = v_rows - tail_w
    if tail_w:
        tail = jnp.concatenate(
            [table[tail_base:].T,
             jnp.zeros((d, 128 - tail_w), jnp.float32)], axis=1)
    else:
        tail = jnp.zeros((d, 128), jnp.float32)

    staging = _build(batch, n_fields, v_rows, d, info.num_cores)(
        xt, offsets.astype(jnp.int32), tbl_t, tail, bias.reshape(-1))
    return staging[:, :d].reshape(batch, n_fields, d)


# speedup vs baseline: 4.0724x; 1.1297x over previous
import functools

import jax
import jax.numpy as jnp
from jax import lax
from jax.experimental import pallas as pl
from jax.experimental.pallas import tpu as pltpu
from jax.experimental.pallas import tpu_sc as plsc

LANES = 16
NBUF = 3
SLABW = 256
SHIFT = 8


def _dsa(start, size, align):
    return pl.ds(pl.multiple_of(start, align), size)


def _splat(x):
    return jnp.broadcast_to(jnp.asarray(x, jnp.int32), (LANES,))


@functools.lru_cache(maxsize=None)
def _build(batch, n_fields, v_rows, d, n_cores):
    total = batch * n_fields
    n_vec = batch // LANES
    hist_n = 512
    d_vecs = d // LANES
    blocks = batch // 128

    mesh = plsc.VectorSubcoreMesh(core_axis_name="c", subcore_axis_name="s")

    @functools.partial(
        pl.kernel,
        mesh=mesh,
        out_type=jax.ShapeDtypeStruct((total, 128), jnp.float32),
        scratch_types=[
            pltpu.VMEM((batch,), jnp.int32),
            pltpu.VMEM((batch,), jnp.int32),
            pltpu.VMEM((batch,), jnp.int32),
            pltpu.VMEM((batch,), jnp.int32),
            pltpu.VMEM((batch,), jnp.int32),
            pltpu.VMEM((hist_n,), jnp.int32),
            pltpu.VMEM((hist_n,), jnp.int32),
            pltpu.VMEM((blocks, 128), jnp.int32),
            pltpu.VMEM((n_fields,), jnp.int32),
            pltpu.VMEM((n_fields * d,), jnp.float32),
            pltpu.VMEM((NBUF, d, SLABW), jnp.float32),
            pltpu.VMEM((2, 128, 128), jnp.float32),
            pltpu.SemaphoreType.DMA,
            pltpu.SemaphoreType.DMA,
            pltpu.SemaphoreType.DMA,
            pltpu.SemaphoreType.DMA,
            pltpu.SemaphoreType.DMA,
        ],
        compiler_params=pltpu.CompilerParams(needs_layout_passes=False),
    )
    def emb_kernel(xt_hbm, off_hbm, tbl_hbm, tail_hbm, bias_hbm, stage_hbm,
                   g_v, s_v, rt_v, srt_g, srt_b, pref_v, run_v, oid_v, off_v,
                   bias_v, slab_v, outb_v, sem0, sem1, sem2, ssem0, ssem1):
        sems = [sem0, sem1, sem2]
        ssems = [ssem0, ssem1]
        cid = lax.axis_index("c")
        sid = lax.axis_index("s")
        wid = sid * n_cores + cid

        @pl.when(wid < n_fields)
        def _worker():
            iota = lax.iota(jnp.int32, LANES)

            pltpu.sync_copy(xt_hbm.at[wid], g_v)
            pltpu.sync_copy(off_hbm, off_v)
            pltpu.sync_copy(bias_hbm, bias_v)

            offv = plsc.load_gather(off_v, [_splat(wid)])
            off_s = jnp.max(offv)
            nxtv = plsc.load_gather(
                off_v, [_splat(jnp.minimum(wid + 1, n_fields - 1))])
            end_g = jnp.where(wid + 1 < n_fields, jnp.max(nxtv),
                              jnp.int32(v_rows))
            first_slab = lax.shift_right_logical(off_s, SHIFT)
            last_slab = lax.shift_right_logical(end_g - 1, SHIFT)
            nslab = last_slab - first_slab + 1
            has_tail = (last_slab + 1) * SLABW > v_rows

            bvs = [plsc.load_gather(bias_v, [_splat(wid * d) + iota + jb * LANES])
                   for jb in range(d_vecs)]

            def prep(i, c):
                sl = _dsa(i * LANES, LANES, 8)
                g = g_v[sl] + offv
                g_v[sl] = g
                s_v[sl] = lax.shift_right_logical(g, SHIFT) - first_slab
                return c
            lax.fori_loop(0, n_vec, prep, 0)

            for k in range(hist_n // LANES):
                pref_v[pl.ds(k * LANES, LANES)] = jnp.zeros((LANES,), jnp.int32)

            def _rank_total(i):
                sl = _dsa(i * LANES, LANES, 8)
                sv = s_v[sl]
                rank = jnp.zeros((LANES,), jnp.int32)
                total_c = jnp.ones((LANES,), jnp.int32)
                hi_idx = jnp.int32(batch - 1)
                for k in range(1, LANES):
                    mb = iota >= k
                    idxb = jnp.clip(_splat(i * LANES - k) + iota, 0, hi_idx)
                    shb = plsc.load_gather(s_v, [idxb], mask=mb)
                    eqb = jnp.where(jnp.logical_and(mb, shb == sv), 1, 0)
                    rank = rank + eqb
                    mf = iota < LANES - k
                    idxf = jnp.clip(_splat(i * LANES + k) + iota, 0, hi_idx)
                    shf = plsc.load_gather(s_v, [idxf], mask=mf)
                    eqf = jnp.where(jnp.logical_and(mf, shf == sv), 1, 0)
                    total_c = total_c + eqb + eqf
                return sv, rank, total_c

            def hist(i, c):
                sv, rank, total_c = _rank_total(i)
                sv = jnp.clip(sv, 0, hist_n - 2)
                base = plsc.load_gather(pref_v, [sv])
                plsc.store_scatter(pref_v, [sv], base + total_c)
                rt_v[_dsa(i * LANES, LANES, 8)] = rank + total_c * LANES
                return c
            lax.fori_loop(0, n_vec, hist, 0)

            carry = jnp.int32(0)
            for k in range(hist_n // LANES):
                sl = pl.ds(k * LANES, LANES)
                h = pref_v[sl]
                inc = plsc.cumsum(h)
                excl = inc - h + jnp.broadcast_to(carry, (LANES,))
                pref_v[sl] = excl
                run_v[sl] = excl
                carry = carry + jnp.max(inc)

            def rank_pass(i, c):
                sl = _dsa(i * LANES, LANES, 8)
                sv = jnp.clip(s_v[sl], 0, hist_n - 2)
                rt = rt_v[sl]
                rank = jnp.bitwise_and(rt, LANES - 1)
                total_c = lax.shift_right_logical(rt, 4)
                base = plsc.load_gather(run_v, [sv])
                slot = jnp.clip(base + rank, 0, batch - 1)
                plsc.store_scatter(srt_g, [slot], g_v[sl])
                plsc.store_scatter(srt_b, [slot], _splat(i * LANES) + iota)
                plsc.store_scatter(run_v, [sv], base + total_c)
                return c
            lax.fori_loop(0, n_vec, rank_pass, 0)

            def post(i, c):
                sl = _dsa(i * LANES, LANES, 8)
                s_v[sl] = jnp.bitwise_and(srt_g[sl], SLABW - 1)
                return c
            lax.fori_loop(0, n_vec, post, 0)

            def oid(j, c):
                for k in range(128 // LANES):
                    bv = srt_b[_dsa(j * 128 + k * LANES, LANES, 8)]
                    oid_v[j, pl.ds(k * LANES, LANES)] = jnp.clip(
                        bv * n_fields + _splat(wid), 0, total - 1)
                return c
            lax.fori_loop(0, blocks, oid, 0)

            def issue(s_idx, p):
                @pl.when(s_idx < nslab)
                def _():
                    is_tail = jnp.logical_and(has_tail, s_idx == nslab - 1)

                    @pl.when(is_tail)
                    def _():
                        pltpu.async_copy(tail_hbm, slab_v.at[p], sems[p])

                    @pl.when(jnp.logical_not(is_tail))
                    def _():
                        col0 = (first_slab + s_idx) * SLABW
                        pltpu.async_copy(
                            tbl_hbm.at[:, _dsa(col0, SLABW, 128)],
                            slab_v.at[p], sems[p])

            for p in range(NBUF):
                issue(jnp.int32(p), p)

            def sweep(t, c):
                for p in range(NBUF):
                    s_idx = t * NBUF + p

                    @pl.when(s_idx < nslab)
                    def _(p=p, s_idx=s_idx):
                        pltpu.make_async_copy(
                            tbl_hbm.at[:, _dsa(0, SLABW, 128)],
                            slab_v.at[p], sems[p]).wait()
                        hi = jnp.minimum(
                            jnp.max(plsc.load_gather(pref_v, [_splat(s_idx + 1)])),
                            jnp.int32(batch))
                        lo = jnp.minimum(
                            jnp.max(plsc.load_gather(pref_v, [_splat(s_idx)])), hi)

                        def occ(q):
                            lanev = plsc.load_gather(
                                s_v, [jnp.clip(_splat(q), 0, batch - 1)])
                            blk = lax.shift_right_logical(q, 7)
                            par = jnp.bitwise_and(blk, 1)
                            row = jnp.bitwise_and(q, 127)

                            for sp in range(2):
                                @pl.when(jnp.logical_and(
                                    jnp.logical_and(row == 0, blk >= 2),
                                    par == sp))
                                def _(sp=sp):
                                    pltpu.make_async_copy(
                                        outb_v.at[sp],
                                        stage_hbm.at[oid_v.at[0]],
                                        ssems[sp]).wait()

                            for jb in range(d_vecs):
                                val = plsc.load_gather(
                                    slab_v.at[p], [iota + jb * LANES, lanev])
                                outb_v[par, row, pl.ds(jb * LANES, LANES)] = \
                                    val + bvs[jb]

                            for sp in range(2):
                                @pl.when(jnp.logical_and(row == 127, par == sp))
                                def _(sp=sp):
                                    pltpu.async_copy(
                                        outb_v.at[sp],
                                        stage_hbm.at[oid_v.at[blk]],
                                        ssems[sp])
                            return q + 1
                        lax.while_loop(lambda q: q < hi, occ, lo)
                        issue(s_idx + NBUF, p)
                return c
            tmax = (nslab + NBUF - 1) // NBUF
            lax.fori_loop(0, tmax, sweep, 0)

            for sp in range(2):
                pltpu.make_async_copy(
                    outb_v.at[sp], stage_hbm.at[oid_v.at[0]], ssems[sp]).wait()

    return emb_kernel


def kernel(x, table, bias, offsets):
    batch, n_fields = x.shape
    v_rows, d = table.shape

    info = plsc.get_sparse_core_info()

    xt = x.T.astype(jnp.int32)
    tbl_t = table.T
    tail_w = v_rows % SLABW
    tail_base = v_rows - tail_w
    if tail_w:
        tail = jnp.concatenate(
            [table[tail_base:].T,
             jnp.zeros((d, SLABW - tail_w), jnp.float32)], axis=1)
    else:
        tail = jnp.zeros((d, SLABW), jnp.float32)

    staging = _build(batch, n_fields, v_rows, d, info.num_cores)(
        xt, offsets.astype(jnp.int32), tbl_t, tail, bias.reshape(-1))
    return staging[:, :d].reshape(batch, n_fields, d)


# speedup vs baseline: 4.1890x; 1.0286x over previous
import functools

import jax
import jax.numpy as jnp
from jax import lax
from jax.experimental import pallas as pl
from jax.experimental.pallas import tpu as pltpu
from jax.experimental.pallas import tpu_sc as plsc

LANES = 16
NBUF = 3
SLABW = 256
SHIFT = 8


def _dsa(start, size, align):
    return pl.ds(pl.multiple_of(start, align), size)


def _splat(x):
    return jnp.broadcast_to(jnp.asarray(x, jnp.int32), (LANES,))


@functools.lru_cache(maxsize=None)
def _build(batch, n_fields, v_rows, d, n_cores):
    total = batch * n_fields
    n_vec = batch // LANES
    hist_n = 512
    d_vecs = d // LANES
    blocks = batch // 128

    mesh = plsc.VectorSubcoreMesh(core_axis_name="c", subcore_axis_name="s")

    @functools.partial(
        pl.kernel,
        mesh=mesh,
        out_type=jax.ShapeDtypeStruct((total, 128), jnp.float32),
        scratch_types=[
            pltpu.VMEM((batch,), jnp.int32),
            pltpu.VMEM((batch,), jnp.int32),
            pltpu.VMEM((batch,), jnp.int32),
            pltpu.VMEM((batch,), jnp.int32),
            pltpu.VMEM((batch,), jnp.int32),
            pltpu.VMEM((hist_n,), jnp.int32),
            pltpu.VMEM((hist_n,), jnp.int32),
            pltpu.VMEM((blocks, 128), jnp.int32),
            pltpu.VMEM((n_fields,), jnp.int32),
            pltpu.VMEM((n_fields * d,), jnp.float32),
            pltpu.VMEM((NBUF, d, SLABW), jnp.float32),
            pltpu.VMEM((2, 128, 128), jnp.float32),
            pltpu.SemaphoreType.DMA,
            pltpu.SemaphoreType.DMA,
            pltpu.SemaphoreType.DMA,
            pltpu.SemaphoreType.DMA,
            pltpu.SemaphoreType.DMA,
        ],
        compiler_params=pltpu.CompilerParams(needs_layout_passes=False),
    )
    def emb_kernel(xt_hbm, off_hbm, tbl_hbm, tail_hbm, bias_hbm, stage_hbm,
                   g_v, s_v, rt_v, srt_g, srt_b, pref_v, run_v, oid_v, off_v,
                   bias_v, slab_v, outb_v, sem0, sem1, sem2, ssem0, ssem1):
        sems = [sem0, sem1, sem2]
        ssems = [ssem0, ssem1]
        cid = lax.axis_index("c")
        sid = lax.axis_index("s")
        wid = sid * n_cores + cid

        @pl.when(wid < n_fields)
        def _worker():
            iota = lax.iota(jnp.int32, LANES)

            pltpu.sync_copy(xt_hbm.at[wid], g_v)
            pltpu.sync_copy(off_hbm, off_v)
            pltpu.sync_copy(bias_hbm, bias_v)

            offv = plsc.load_gather(off_v, [_splat(wid)])
            off_s = jnp.max(offv)
            nxtv = plsc.load_gather(
                off_v, [_splat(jnp.minimum(wid + 1, n_fields - 1))])
            end_g = jnp.where(wid + 1 < n_fields, jnp.max(nxtv),
                              jnp.int32(v_rows))
            first_slab = lax.shift_right_logical(off_s, SHIFT)
            last_slab = lax.shift_right_logical(end_g - 1, SHIFT)
            nslab = last_slab - first_slab + 1
            has_tail = (last_slab + 1) * SLABW > v_rows

            bvs = [plsc.load_gather(bias_v, [_splat(wid * d) + iota + jb * LANES])
                   for jb in range(d_vecs)]

            def prep(i, c):
                sl = _dsa(i * LANES, LANES, 8)
                g = g_v[sl] + offv
                g_v[sl] = g
                s_v[sl] = lax.shift_right_logical(g, SHIFT) - first_slab
                return c
            lax.fori_loop(0, n_vec, prep, 0)

            for k in range(hist_n // LANES):
                pref_v[pl.ds(k * LANES, LANES)] = jnp.zeros((LANES,), jnp.int32)

            def hist(i, c):
                sv = jnp.clip(s_v[_dsa(i * LANES, LANES, 8)], 0, hist_n - 2)
                r, m = plsc.scan_count(sv)
                base = plsc.load_gather(pref_v, [sv])
                plsc.store_scatter(pref_v, [sv], base + r, mask=m)
                return c
            lax.fori_loop(0, n_vec, hist, 0)

            carry = jnp.int32(0)
            for k in range(hist_n // LANES):
                sl = pl.ds(k * LANES, LANES)
                h = pref_v[sl]
                inc = plsc.cumsum(h)
                excl = inc - h + jnp.broadcast_to(carry, (LANES,))
                pref_v[sl] = excl
                run_v[sl] = excl
                carry = carry + jnp.max(inc)

            def rank_pass(i, c):
                sl = _dsa(i * LANES, LANES, 8)
                sv = jnp.clip(s_v[sl], 0, hist_n - 2)
                r, m = plsc.scan_count(sv)
                base = plsc.load_gather(run_v, [sv])
                slot = jnp.clip(base + r - 1, 0, batch - 1)
                plsc.store_scatter(srt_g, [slot], g_v[sl])
                plsc.store_scatter(srt_b, [slot], _splat(i * LANES) + iota)
                plsc.store_scatter(run_v, [sv], base + r, mask=m)
                return c
            lax.fori_loop(0, n_vec, rank_pass, 0)

            def post(i, c):
                sl = _dsa(i * LANES, LANES, 8)
                s_v[sl] = jnp.bitwise_and(srt_g[sl], SLABW - 1)
                return c
            lax.fori_loop(0, n_vec, post, 0)

            def oid(j, c):
                for k in range(128 // LANES):
                    bv = srt_b[_dsa(j * 128 + k * LANES, LANES, 8)]
                    oid_v[j, pl.ds(k * LANES, LANES)] = jnp.clip(
                        bv * n_fields + _splat(wid), 0, total - 1)
                return c
            lax.fori_loop(0, blocks, oid, 0)

            def issue(s_idx, p):
                @pl.when(s_idx < nslab)
                def _():
                    is_tail = jnp.logical_and(has_tail, s_idx == nslab - 1)

                    @pl.when(is_tail)
                    def _():
                        pltpu.async_copy(tail_hbm, slab_v.at[p], sems[p])

                    @pl.when(jnp.logical_not(is_tail))
                    def _():
                        col0 = (first_slab + s_idx) * SLABW
                        pltpu.async_copy(
                            tbl_hbm.at[:, _dsa(col0, SLABW, 128)],
                            slab_v.at[p], sems[p])

            for p in range(NBUF):
                issue(jnp.int32(p), p)

            def sweep(t, c):
                for p in range(NBUF):
                    s_idx = t * NBUF + p

                    @pl.when(s_idx < nslab)
                    def _(p=p, s_idx=s_idx):
                        pltpu.make_async_copy(
                            tbl_hbm.at[:, _dsa(0, SLABW, 128)],
                            slab_v.at[p], sems[p]).wait()
                        hi = jnp.minimum(
                            jnp.max(plsc.load_gather(pref_v, [_splat(s_idx + 1)])),
                            jnp.int32(batch))
                        lo = jnp.minimum(
                            jnp.max(plsc.load_gather(pref_v, [_splat(s_idx)])), hi)

                        def occ(q):
                            lanev = plsc.load_gather(
                                s_v, [jnp.clip(_splat(q), 0, batch - 1)])
                            blk = lax.shift_right_logical(q, 7)
                            par = jnp.bitwise_and(blk, 1)
                            row = jnp.bitwise_and(q, 127)

                            for sp in range(2):
                                @pl.when(jnp.logical_and(
                                    jnp.logical_and(row == 0, blk >= 2),
                                    par == sp))
                                def _(sp=sp):
                                    pltpu.make_async_copy(
                                        outb_v.at[sp],
                                        stage_hbm.at[oid_v.at[0]],
                                        ssems[sp]).wait()

                            for jb in range(d_vecs):
                                val = plsc.load_gather(
                                    slab_v.at[p], [iota + jb * LANES, lanev])
                                outb_v[par, row, pl.ds(jb * LANES, LANES)] = \
                                    val + bvs[jb]

                            for sp in range(2):
                                @pl.when(jnp.logical_and(row == 127, par == sp))
                                def _(sp=sp):
                                    pltpu.async_copy(
                                        outb_v.at[sp],
                                        stage_hbm.at[oid_v.at[blk]],
                                        ssems[sp])
                            return q + 1
                        lax.while_loop(lambda q: q < hi, occ, lo)
                        issue(s_idx + NBUF, p)
                return c
            tmax = (nslab + NBUF - 1) // NBUF
            lax.fori_loop(0, tmax, sweep, 0)

            for sp in range(2):
                pltpu.make_async_copy(
                    outb_v.at[sp], stage_hbm.at[oid_v.at[0]], ssems[sp]).wait()

    return emb_kernel


def kernel(x, table, bias, offsets):
    batch, n_fields = x.shape
    v_rows, d = table.shape

    info = plsc.get_sparse_core_info()

    xt = x.T.astype(jnp.int32)
    tbl_t = table.T
    tail_w = v_rows % SLABW
    tail_base = v_rows - tail_w
    if tail_w:
        tail = jnp.concatenate(
            [table[tail_base:].T,
             jnp.zeros((d, SLABW - tail_w), jnp.float32)], axis=1)
    else:
        tail = jnp.zeros((d, SLABW), jnp.float32)

    staging = _build(batch, n_fields, v_rows, d, info.num_cores)(
        xt, offsets.astype(jnp.int32), tbl_t, tail, bias.reshape(-1))
    return staging[:, :d].reshape(batch, n_fields, d)


# speedup vs baseline: 4.4847x; 1.0706x over previous
import functools

import jax
import jax.numpy as jnp
from jax import lax
from jax.experimental import pallas as pl
from jax.experimental.pallas import tpu as pltpu
from jax.experimental.pallas import tpu_sc as plsc

LANES = 16
NBUF = 4
SLABW = 256
SHIFT = 8


def _dsa(start, size, align):
    return pl.ds(pl.multiple_of(start, align), size)


def _splat(x):
    return jnp.broadcast_to(jnp.asarray(x, jnp.int32), (LANES,))


@functools.lru_cache(maxsize=None)
def _build(batch, n_fields, v_rows, d, n_cores):
    total = batch * n_fields
    n_vec = batch // LANES
    hist_n = 512
    d_vecs = d // LANES
    blocks = batch // 128

    mesh = plsc.VectorSubcoreMesh(core_axis_name="c", subcore_axis_name="s")

    @functools.partial(
        pl.kernel,
        mesh=mesh,
        out_type=jax.ShapeDtypeStruct((total, 128), jnp.float32),
        scratch_types=[
            pltpu.VMEM((batch,), jnp.int32),
            pltpu.VMEM((batch,), jnp.int32),
            pltpu.VMEM((batch,), jnp.int32),
            pltpu.VMEM((batch,), jnp.int32),
            pltpu.VMEM((batch,), jnp.int32),
            pltpu.VMEM((hist_n,), jnp.int32),
            pltpu.VMEM((hist_n,), jnp.int32),
            pltpu.VMEM((blocks, 128), jnp.int32),
            pltpu.VMEM((n_fields,), jnp.int32),
            pltpu.VMEM((n_fields * d,), jnp.float32),
            pltpu.VMEM((NBUF, d, SLABW), jnp.float32),
            pltpu.VMEM((2, 128, 128), jnp.float32),
            pltpu.SemaphoreType.DMA,
            pltpu.SemaphoreType.DMA,
            pltpu.SemaphoreType.DMA,
            pltpu.SemaphoreType.DMA,
            pltpu.SemaphoreType.DMA,
            pltpu.SemaphoreType.DMA,
        ],
        compiler_params=pltpu.CompilerParams(needs_layout_passes=False),
    )
    def emb_kernel(xt_hbm, off_hbm, tbl_hbm, tail_hbm, bias_hbm, stage_hbm,
                   g_v, s_v, rt_v, srt_g, srt_b, pref_v, run_v, oid_v, off_v,
                   bias_v, slab_v, outb_v, sem0, sem1, sem2, sem3, ssem0,
                   ssem1):
        sems = [sem0, sem1, sem2, sem3]
        ssems = [ssem0, ssem1]
        cid = lax.axis_index("c")
        sid = lax.axis_index("s")
        wid = sid * n_cores + cid

        @pl.when(wid < n_fields)
        def _worker():
            iota = lax.iota(jnp.int32, LANES)

            pltpu.sync_copy(xt_hbm.at[wid], g_v)
            pltpu.sync_copy(off_hbm, off_v)
            pltpu.sync_copy(bias_hbm, bias_v)

            offv = plsc.load_gather(off_v, [_splat(wid)])
            off_s = jnp.max(offv)
            nxtv = plsc.load_gather(
                off_v, [_splat(jnp.minimum(wid + 1, n_fields - 1))])
            end_g = jnp.where(wid + 1 < n_fields, jnp.max(nxtv),
                              jnp.int32(v_rows))
            first_slab = lax.shift_right_logical(off_s, SHIFT)
            last_slab = lax.shift_right_logical(end_g - 1, SHIFT)
            nslab = last_slab - first_slab + 1
            has_tail = (last_slab + 1) * SLABW > v_rows

            bvs = [plsc.load_gather(bias_v, [_splat(wid * d) + iota + jb * LANES])
                   for jb in range(d_vecs)]

            def prep(i, c):
                sl = _dsa(i * LANES, LANES, 8)
                g = g_v[sl] + offv
                g_v[sl] = g
                s_v[sl] = lax.shift_right_logical(g, SHIFT) - first_slab
                return c
            lax.fori_loop(0, n_vec, prep, 0)

            for k in range(hist_n // LANES):
                pref_v[pl.ds(k * LANES, LANES)] = jnp.zeros((LANES,), jnp.int32)

            def hist(i, c):
                sv = jnp.clip(s_v[_dsa(i * LANES, LANES, 8)], 0, hist_n - 2)
                r, m = plsc.scan_count(sv)
                base = plsc.load_gather(pref_v, [sv])
                plsc.store_scatter(pref_v, [sv], base + r, mask=m)
                return c
            lax.fori_loop(0, n_vec, hist, 0)

            carry = jnp.int32(0)
            for k in range(hist_n // LANES):
                sl = pl.ds(k * LANES, LANES)
                h = pref_v[sl]
                inc = plsc.cumsum(h)
                excl = inc - h + jnp.broadcast_to(carry, (LANES,))
                pref_v[sl] = excl
                run_v[sl] = excl
                carry = carry + jnp.max(inc)

            def rank_pass(i, c):
                sl = _dsa(i * LANES, LANES, 8)
                sv = jnp.clip(s_v[sl], 0, hist_n - 2)
                r, m = plsc.scan_count(sv)
                base = plsc.load_gather(run_v, [sv])
                slot = jnp.clip(base + r - 1, 0, batch - 1)
                plsc.store_scatter(srt_g, [slot], g_v[sl])
                plsc.store_scatter(srt_b, [slot], _splat(i * LANES) + iota)
                plsc.store_scatter(run_v, [sv], base + r, mask=m)
                return c
            lax.fori_loop(0, n_vec, rank_pass, 0)

            def post(i, c):
                sl = _dsa(i * LANES, LANES, 8)
                s_v[sl] = jnp.bitwise_and(srt_g[sl], SLABW - 1)
                return c
            lax.fori_loop(0, n_vec, post, 0)

            def oid(j, c):
                for k in range(128 // LANES):
                    bv = srt_b[_dsa(j * 128 + k * LANES, LANES, 8)]
                    oid_v[j, pl.ds(k * LANES, LANES)] = jnp.clip(
                        bv * n_fields + _splat(wid), 0, total - 1)
                return c
            lax.fori_loop(0, blocks, oid, 0)

            def issue(s_idx, p):
                @pl.when(s_idx < nslab)
                def _():
                    is_tail = jnp.logical_and(has_tail, s_idx == nslab - 1)

                    @pl.when(is_tail)
                    def _():
                        pltpu.async_copy(tail_hbm, slab_v.at[p], sems[p])

                    @pl.when(jnp.logical_not(is_tail))
                    def _():
                        col0 = (first_slab + s_idx) * SLABW
                        pltpu.async_copy(
                            tbl_hbm.at[:, _dsa(col0, SLABW, 128)],
                            slab_v.at[p], sems[p])

            for p in range(NBUF):
                issue(jnp.int32(p), p)

            def sweep(t, c):
                for p in range(NBUF):
                    s_idx = t * NBUF + p

                    @pl.when(s_idx < nslab)
                    def _(p=p, s_idx=s_idx):
                        pltpu.make_async_copy(
                            tbl_hbm.at[:, _dsa(0, SLABW, 128)],
                            slab_v.at[p], sems[p]).wait()
                        hi = jnp.minimum(
                            jnp.max(plsc.load_gather(pref_v, [_splat(s_idx + 1)])),
                            jnp.int32(batch))
                        lo = jnp.minimum(
                            jnp.max(plsc.load_gather(pref_v, [_splat(s_idx)])), hi)

                        def occ(q):
                            lanev = plsc.load_gather(
                                s_v, [jnp.clip(_splat(q), 0, batch - 1)])
                            blk = lax.shift_right_logical(q, 7)
                            par = jnp.bitwise_and(blk, 1)
                            row = jnp.bitwise_and(q, 127)

                            for sp in range(2):
                                @pl.when(jnp.logical_and(
                                    jnp.logical_and(row == 0, blk >= 2),
                                    par == sp))
                                def _(sp=sp):
                                    pltpu.make_async_copy(
                                        outb_v.at[sp],
                                        stage_hbm.at[oid_v.at[0]],
                                        ssems[sp]).wait()

                            for jb in range(d_vecs):
                                val = plsc.load_gather(
                                    slab_v.at[p], [iota + jb * LANES, lanev])
                                outb_v[par, row, pl.ds(jb * LANES, LANES)] = \
                                    val + bvs[jb]

                            for sp in range(2):
                                @pl.when(jnp.logical_and(row == 127, par == sp))
                                def _(sp=sp):
                                    pltpu.async_copy(
                                        outb_v.at[sp],
                                        stage_hbm.at[oid_v.at[blk]],
                                        ssems[sp])
                            return q + 1
                        lax.while_loop(lambda q: q < hi, occ, lo)
                        issue(s_idx + NBUF, p)
                return c
            tmax = (nslab + NBUF - 1) // NBUF
            lax.fori_loop(0, tmax, sweep, 0)

            for sp in range(2):
                pltpu.make_async_copy(
                    outb_v.at[sp], stage_hbm.at[oid_v.at[0]], ssems[sp]).wait()

    return emb_kernel


def kernel(x, table, bias, offsets):
    batch, n_fields = x.shape
    v_rows, d = table.shape

    info = plsc.get_sparse_core_info()

    xt = x.T.astype(jnp.int32)
    tbl_t = table.T
    tail_w = v_rows % SLABW
    tail_base = v_rows - tail_w
    if tail_w:
        tail = jnp.concatenate(
            [table[tail_base:].T,
             jnp.zeros((d, SLABW - tail_w), jnp.float32)], axis=1)
    else:
        tail = jnp.zeros((d, SLABW), jnp.float32)

    staging = _build(batch, n_fields, v_rows, d, info.num_cores)(
        xt, offsets.astype(jnp.int32), tbl_t, tail, bias.reshape(-1))
    return staging[:, :d].reshape(batch, n_fields, d)
